# baseline (speedup 1.0000x reference)
import jax
import jax.numpy as jnp
from jax.experimental import pallas as pl
from jax.experimental.pallas import tpu as pltpu

INTERVAL = 0.1
ZETA = 0.005
ALPHA = 0.5
NUM_NB = 8
W_SC = 1.0
W_SS = 1.0
W_RD = 1.0

B = 8
N = 2048
R = 2048
NB = N // R
BIG = 1e30
DTH = ZETA * 2.5 * N


def _loss_kernel(pc1_ref, pc2_ref, pf_ref, pc1t_ref, pft_ref, vel_ref,
                 out_ref, colmin_ref, colsum_ref, acc_ref):
    i = pl.program_id(1)
    first = (pl.program_id(0) == 0) & (i == 0)

    @pl.when(first)
    def _():
        out_ref[0, 0] = 0.0

    p_cols = pc1_ref[0]
    q_cols = pc2_ref[0]
    f_cols = pf_ref[0]
    prt = pc1t_ref[0, pl.ds(i * R, R), :]
    frt = pft_ref[0, pl.ds(i * R, R), :]

    def sqdist(rows_t, cols):
        acc = None
        for c in range(3):
            diff = rows_t[:, c:c + 1] - cols[c:c + 1, :]
            t = diff * diff
            acc = t if acc is None else acc + t
        return acc

    @pl.when(i == 0)
    def _():
        vel = vel_ref[0]
        fdotp = jnp.sum(f_cols * p_cols, axis=0, keepdims=True)
        pn = jnp.sqrt(jnp.sum(p_cols * p_cols, axis=0, keepdims=True))
        rd = jnp.sum(jnp.abs(vel * INTERVAL - fdotp / pn))
        out_ref[0, 0] += (W_RD / (B * N)) * rd
        acc_ref[0] = 0.0
        acc_ref[1] = 0.0

    d1 = sqdist(prt, q_cols)
    g = jnp.exp(d1 * -0.5)
    dens12 = jnp.sum(g, axis=1, keepdims=True)
    gcol = jnp.sum(g, axis=0, keepdims=True)

    dw = sqdist(prt + frt, q_cols)
    min1 = jnp.min(dw, axis=1, keepdims=True)
    cmin = jnp.min(dw, axis=0, keepdims=True)

    @pl.when(i == 0)
    def _():
        colmin_ref[...] = cmin
        colsum_ref[...] = gcol

    @pl.when(i > 0)
    def _():
        colmin_ref[...] = jnp.minimum(colmin_ref[...], cmin)
        colsum_ref[...] = colsum_ref[...] + gcol

    c1 = jnp.sum(jnp.where(dens12 > DTH, jnp.maximum(min1 - 0.01, 0.0), 0.0))
    out_ref[0, 0] += (W_SC / (B * N)) * c1

    ds = sqdist(prt, p_cols)
    fd = jnp.sqrt(sqdist(frt, f_cols))
    iota_j = jax.lax.broadcasted_iota(jnp.int32, (R, N), 1)
    row_ids = jax.lax.broadcasted_iota(jnp.int32, (R, 1), 0) + i * R
    keys_i = (jax.lax.bitcast_convert_type(ds, jnp.int32) & jnp.int32(~0x7FF)
              ) | iota_j
    keys_i = jnp.where(iota_j == row_ids, jnp.int32(0x7F800000), keys_i)
    keys = jax.lax.bitcast_convert_type(keys_i, jnp.float32)

    m = jnp.min(keys, axis=1, keepdims=True)
    for _ in range(NUM_NB - 1):
        m = jnp.min(jnp.where(keys > m, keys, jnp.inf),
                    axis=1, keepdims=True)
    sel = keys <= m
    wsel = jnp.where(sel, jnp.exp(jnp.exp(ds * (-1.0 / ALPHA))), 0.0)
    acc_ref[0] += jnp.sum(wsel)
    acc_ref[1] += jnp.sum(wsel * fd)

    @pl.when(i == NB - 1)
    def _():
        c2 = jnp.sum(jnp.where(colsum_ref[...] > DTH,
                               jnp.maximum(colmin_ref[...] - 0.01, 0.0), 0.0))
        out_ref[0, 0] += (W_SC / (B * N)) * c2
        out_ref[0, 0] += (W_SS / B) * acc_ref[1] / acc_ref[0]


@jax.jit
def kernel(pc1, pc2, pred_f, vel1):
    pc1t = jnp.transpose(pc1, (0, 2, 1))
    pft = jnp.transpose(pred_f, (0, 2, 1))
    velr = vel1[:, None, :]
    out = pl.pallas_call(
        _loss_kernel,
        grid=(B, NB),
        in_specs=[
            pl.BlockSpec((1, 3, N), lambda b, i: (b, 0, 0)),
            pl.BlockSpec((1, 3, N), lambda b, i: (b, 0, 0)),
            pl.BlockSpec((1, 3, N), lambda b, i: (b, 0, 0)),
            pl.BlockSpec((1, N, 3), lambda b, i: (b, 0, 0)),
            pl.BlockSpec((1, N, 3), lambda b, i: (b, 0, 0)),
            pl.BlockSpec((1, 1, N), lambda b, i: (b, 0, 0)),
        ],
        out_specs=pl.BlockSpec(memory_space=pltpu.SMEM),
        out_shape=jax.ShapeDtypeStruct((1, 1), jnp.float32),
        scratch_shapes=[
            pltpu.VMEM((1, N), jnp.float32),
            pltpu.VMEM((1, N), jnp.float32),
            pltpu.SMEM((2,), jnp.float32),
        ],
    )(pc1, pc2, pred_f, pc1t, pft, velr)
    return out[0, 0]


# speedup vs baseline: 1.2637x; 1.2637x over previous
import jax
import jax.numpy as jnp
from jax.experimental import pallas as pl
from jax.experimental.pallas import tpu as pltpu

INTERVAL = 0.1
ZETA = 0.005
ALPHA = 0.5
NUM_NB = 8
W_SC = 1.0
W_SS = 1.0
W_RD = 1.0

B = 8
N = 2048
R = 1024
NB = N // R
BIG = 1e30
DTH = ZETA * 2.5 * N


def _loss_kernel(pc1_ref, pc2_ref, pf_ref, pc1t_ref, pft_ref, vel_ref,
                 out_ref, colmin_ref, colsum_ref, acc_ref):
    i = pl.program_id(1)
    first = (pl.program_id(0) == 0) & (i == 0)

    @pl.when(first)
    def _():
        out_ref[0, 0] = 0.0

    p_cols = pc1_ref[0]
    q_cols = pc2_ref[0]
    f_cols = pf_ref[0]
    prt = pc1t_ref[0, pl.ds(i * R, R), :]
    frt = pft_ref[0, pl.ds(i * R, R), :]

    def sqdist(rows_t, cols):
        acc = None
        for c in range(3):
            diff = rows_t[:, c:c + 1] - cols[c:c + 1, :]
            t = diff * diff
            acc = t if acc is None else acc + t
        return acc

    @pl.when(i == 0)
    def _():
        vel = vel_ref[0]
        fdotp = jnp.sum(f_cols * p_cols, axis=0, keepdims=True)
        pn = jnp.sqrt(jnp.sum(p_cols * p_cols, axis=0, keepdims=True))
        rd = jnp.sum(jnp.abs(vel * INTERVAL - fdotp / pn))
        out_ref[0, 0] += (W_RD / (B * N)) * rd
        acc_ref[0] = 0.0
        acc_ref[1] = 0.0

    d1 = sqdist(prt, q_cols)
    g = jnp.exp(d1 * -0.5)
    dens12 = jnp.sum(g, axis=1, keepdims=True)
    gcol = jnp.sum(g, axis=0, keepdims=True)

    dw = sqdist(prt + frt, q_cols)
    min1 = jnp.min(dw, axis=1, keepdims=True)
    cmin = jnp.min(dw, axis=0, keepdims=True)

    @pl.when(i == 0)
    def _():
        colmin_ref[...] = cmin
        colsum_ref[...] = gcol

    @pl.when(i > 0)
    def _():
        colmin_ref[...] = jnp.minimum(colmin_ref[...], cmin)
        colsum_ref[...] = colsum_ref[...] + gcol

    c1 = jnp.sum(jnp.where(dens12 > DTH, jnp.maximum(min1 - 0.01, 0.0), 0.0))
    out_ref[0, 0] += (W_SC / (B * N)) * c1

    ds = sqdist(prt, p_cols)
    fd2 = sqdist(frt, f_cols)
    fd = fd2 * jax.lax.rsqrt(fd2 + 1e-30)
    iota_j = jax.lax.broadcasted_iota(jnp.int32, (R, N), 1)
    row_ids = jax.lax.broadcasted_iota(jnp.int32, (R, 1), 0) + i * R
    keys_i = (jax.lax.bitcast_convert_type(ds, jnp.int32) & jnp.int32(~0x7FF)
              ) | iota_j
    keys_i = jnp.where(iota_j == row_ids, jnp.int32(0x7F800000), keys_i)
    keys = jax.lax.bitcast_convert_type(keys_i, jnp.float32)

    m = jnp.min(keys, axis=1, keepdims=True)
    for _ in range(NUM_NB - 1):
        m = jnp.min(jnp.where(keys > m, keys, jnp.inf),
                    axis=1, keepdims=True)
    sel = keys <= m
    wsel = jnp.where(sel, jnp.exp(jnp.exp(ds * (-1.0 / ALPHA))), 0.0)
    acc_ref[0] += jnp.sum(wsel)
    acc_ref[1] += jnp.sum(wsel * fd)

    @pl.when(i == NB - 1)
    def _():
        c2 = jnp.sum(jnp.where(colsum_ref[...] > DTH,
                               jnp.maximum(colmin_ref[...] - 0.01, 0.0), 0.0))
        out_ref[0, 0] += (W_SC / (B * N)) * c2
        out_ref[0, 0] += (W_SS / B) * acc_ref[1] / acc_ref[0]


@jax.jit
def kernel(pc1, pc2, pred_f, vel1):
    pc1t = jnp.transpose(pc1, (0, 2, 1))
    pft = jnp.transpose(pred_f, (0, 2, 1))
    velr = vel1[:, None, :]
    out = pl.pallas_call(
        _loss_kernel,
        grid=(B, NB),
        in_specs=[
            pl.BlockSpec((1, 3, N), lambda b, i: (b, 0, 0)),
            pl.BlockSpec((1, 3, N), lambda b, i: (b, 0, 0)),
            pl.BlockSpec((1, 3, N), lambda b, i: (b, 0, 0)),
            pl.BlockSpec((1, N, 3), lambda b, i: (b, 0, 0)),
            pl.BlockSpec((1, N, 3), lambda b, i: (b, 0, 0)),
            pl.BlockSpec((1, 1, N), lambda b, i: (b, 0, 0)),
        ],
        out_specs=pl.BlockSpec(memory_space=pltpu.SMEM),
        out_shape=jax.ShapeDtypeStruct((1, 1), jnp.float32),
        scratch_shapes=[
            pltpu.VMEM((1, N), jnp.float32),
            pltpu.VMEM((1, N), jnp.float32),
            pltpu.SMEM((2,), jnp.float32),
        ],
    )(pc1, pc2, pred_f, pc1t, pft, velr)
    return out[0, 0]


# speedup vs baseline: 1.2911x; 1.0217x over previous
import jax
import jax.numpy as jnp
from jax.experimental import pallas as pl
from jax.experimental.pallas import tpu as pltpu

INTERVAL = 0.1
ZETA = 0.005
ALPHA = 0.5
NUM_NB = 8
W_SC = 1.0
W_SS = 1.0
W_RD = 1.0

B = 8
N = 2048
R = 1024
NB = N // R
BIG = 1e30
DTH = ZETA * 2.5 * N


def _loss_kernel(pc1_ref, pc2_ref, pf_ref, pc1t_ref, pft_ref, vel_ref,
                 out_ref, colmin_ref, colsum_ref, acc_ref):
    i = pl.program_id(1)
    first = (pl.program_id(0) == 0) & (i == 0)

    @pl.when(first)
    def _():
        out_ref[0, 0] = 0.0

    p_cols = pc1_ref[0]
    q_cols = pc2_ref[0]
    f_cols = pf_ref[0]
    prt = pc1t_ref[0, pl.ds(i * R, R), :]
    frt = pft_ref[0, pl.ds(i * R, R), :]

    def sqdist(rows_t, cols):
        acc = None
        for c in range(3):
            diff = rows_t[:, c:c + 1] - cols[c:c + 1, :]
            t = diff * diff
            acc = t if acc is None else acc + t
        return acc

    @pl.when(i == 0)
    def _():
        vel = vel_ref[0]
        fdotp = jnp.sum(f_cols * p_cols, axis=0, keepdims=True)
        pn = jnp.sqrt(jnp.sum(p_cols * p_cols, axis=0, keepdims=True))
        rd = jnp.sum(jnp.abs(vel * INTERVAL - fdotp / pn))
        out_ref[0, 0] += (W_RD / (B * N)) * rd
        acc_ref[0] = 0.0
        acc_ref[1] = 0.0

    d1 = sqdist(prt, q_cols)
    g = jnp.exp(d1 * -0.5)
    dens12 = jnp.sum(g, axis=1, keepdims=True)
    gcol = jnp.sum(g, axis=0, keepdims=True)

    dw = sqdist(prt + frt, q_cols)
    min1 = jnp.min(dw, axis=1, keepdims=True)
    cmin = jnp.min(dw, axis=0, keepdims=True)

    @pl.when(i == 0)
    def _():
        colmin_ref[...] = cmin
        colsum_ref[...] = gcol

    @pl.when(i > 0)
    def _():
        colmin_ref[...] = jnp.minimum(colmin_ref[...], cmin)
        colsum_ref[...] = colsum_ref[...] + gcol

    c1 = jnp.sum(jnp.where(dens12 > DTH, jnp.maximum(min1 - 0.01, 0.0), 0.0))
    out_ref[0, 0] += (W_SC / (B * N)) * c1

    ds = sqdist(prt, p_cols)
    fd2 = sqdist(frt, f_cols)
    fd = fd2 * jax.lax.rsqrt(fd2 + 1e-30)
    iota_j = jax.lax.broadcasted_iota(jnp.int32, (R, N), 1)
    row_ids = jax.lax.broadcasted_iota(jnp.int32, (R, 1), 0) + i * R
    keys = jnp.where(iota_j == row_ids, jnp.inf, ds)

    m = jnp.min(keys, axis=1, keepdims=True)
    for _ in range(NUM_NB - 1):
        m = jnp.min(jnp.where(keys > m, keys, jnp.inf),
                    axis=1, keepdims=True)
    sel = keys <= m
    wsel = jnp.where(sel, jnp.exp(jnp.exp(ds * (-1.0 / ALPHA))), 0.0)
    acc_ref[0] += jnp.sum(wsel)
    acc_ref[1] += jnp.sum(wsel * fd)

    @pl.when(i == NB - 1)
    def _():
        c2 = jnp.sum(jnp.where(colsum_ref[...] > DTH,
                               jnp.maximum(colmin_ref[...] - 0.01, 0.0), 0.0))
        out_ref[0, 0] += (W_SC / (B * N)) * c2
        out_ref[0, 0] += (W_SS / B) * acc_ref[1] / acc_ref[0]


@jax.jit
def kernel(pc1, pc2, pred_f, vel1):
    pc1t = jnp.transpose(pc1, (0, 2, 1))
    pft = jnp.transpose(pred_f, (0, 2, 1))
    velr = vel1[:, None, :]
    out = pl.pallas_call(
        _loss_kernel,
        grid=(B, NB),
        in_specs=[
            pl.BlockSpec((1, 3, N), lambda b, i: (b, 0, 0)),
            pl.BlockSpec((1, 3, N), lambda b, i: (b, 0, 0)),
            pl.BlockSpec((1, 3, N), lambda b, i: (b, 0, 0)),
            pl.BlockSpec((1, N, 3), lambda b, i: (b, 0, 0)),
            pl.BlockSpec((1, N, 3), lambda b, i: (b, 0, 0)),
            pl.BlockSpec((1, 1, N), lambda b, i: (b, 0, 0)),
        ],
        out_specs=pl.BlockSpec(memory_space=pltpu.SMEM),
        out_shape=jax.ShapeDtypeStruct((1, 1), jnp.float32),
        scratch_shapes=[
            pltpu.VMEM((1, N), jnp.float32),
            pltpu.VMEM((1, N), jnp.float32),
            pltpu.SMEM((2,), jnp.float32),
        ],
    )(pc1, pc2, pred_f, pc1t, pft, velr)
    return out[0, 0]
